# baseline (speedup 1.0000x reference)
import jax
import jax.numpy as jnp
from jax.experimental import pallas as pl
from jax.experimental.pallas import tpu as pltpu

_CHUNKS = 10


def _copy_kernel(x_ref, o_ref, vmem, in_sems, out_sems):
    n = x_ref.shape[0]
    rows = n // _CHUNKS
    ins = []
    for i in range(_CHUNKS):
        s = jnp.int32(i * rows)
        c = pltpu.make_async_copy(
            x_ref.at[pl.ds(s, rows), :],
            vmem.at[pl.ds(s, rows), :],
            in_sems.at[jnp.int32(i)],
        )
        c.start()
        ins.append(c)
    outs = []
    for i in range(_CHUNKS):
        ins[i].wait()
        s = jnp.int32(i * rows)
        c = pltpu.make_async_copy(
            vmem.at[pl.ds(s, rows), :],
            o_ref.at[pl.ds(s, rows), :],
            out_sems.at[jnp.int32(i)],
        )
        c.start()
        outs.append(c)
    for c in outs:
        c.wait()


def kernel(x, edge_index):
    del edge_index
    n, d = x.shape
    return pl.pallas_call(
        _copy_kernel,
        in_specs=[pl.BlockSpec(memory_space=pltpu.MemorySpace.HBM)],
        out_specs=pl.BlockSpec(memory_space=pltpu.MemorySpace.HBM),
        out_shape=jax.ShapeDtypeStruct((n, d), x.dtype),
        scratch_shapes=[
            pltpu.VMEM((n, d), x.dtype),
            pltpu.SemaphoreType.DMA((_CHUNKS,)),
            pltpu.SemaphoreType.DMA((_CHUNKS,)),
        ],
    )(x)


# speedup vs baseline: 1.0057x; 1.0057x over previous
import jax
import jax.numpy as jnp
from jax.experimental import pallas as pl

_BLOCK_ROWS = 5000


def _copy_block(x_ref, o_ref):
    o_ref[...] = x_ref[...]


def kernel(x, edge_index):
    del edge_index
    n, d = x.shape
    return pl.pallas_call(
        _copy_block,
        grid=(n // _BLOCK_ROWS,),
        in_specs=[pl.BlockSpec((_BLOCK_ROWS, d), lambda i: (i, jnp.int32(0)))],
        out_specs=pl.BlockSpec((_BLOCK_ROWS, d), lambda i: (i, jnp.int32(0))),
        out_shape=jax.ShapeDtypeStruct((n, d), x.dtype),
    )(x)


# speedup vs baseline: 1.0413x; 1.0353x over previous
import jax
import jax.numpy as jnp
from jax.experimental import pallas as pl
from jax.experimental.pallas import tpu as pltpu

_CHUNKS = 5


def _copy_kernel(x_ref, o_ref, vmem, in_sems, out_sems):
    n = x_ref.shape[0]
    rows = n // _CHUNKS
    ins = []
    for i in range(_CHUNKS):
        s = jnp.int32(i * rows)
        c = pltpu.make_async_copy(
            x_ref.at[pl.ds(s, rows), :],
            vmem.at[pl.ds(s, rows), :],
            in_sems.at[jnp.int32(i)],
        )
        c.start()
        ins.append(c)
    outs = []
    for i in range(_CHUNKS):
        ins[i].wait()
        s = jnp.int32(i * rows)
        c = pltpu.make_async_copy(
            vmem.at[pl.ds(s, rows), :],
            o_ref.at[pl.ds(s, rows), :],
            out_sems.at[jnp.int32(i)],
        )
        c.start()
        outs.append(c)
    for c in outs:
        c.wait()


def kernel(x, edge_index):
    del edge_index
    n, d = x.shape
    return pl.pallas_call(
        _copy_kernel,
        in_specs=[pl.BlockSpec(memory_space=pltpu.MemorySpace.HBM)],
        out_specs=pl.BlockSpec(memory_space=pltpu.MemorySpace.HBM),
        out_shape=jax.ShapeDtypeStruct((n, d), x.dtype),
        scratch_shapes=[
            pltpu.VMEM((n, d), x.dtype),
            pltpu.SemaphoreType.DMA((_CHUNKS,)),
            pltpu.SemaphoreType.DMA((_CHUNKS,)),
        ],
    )(x)
